# Initial kernel scaffold; baseline (speedup 1.0000x reference)
import jax
import jax.numpy as jnp
from jax.experimental import pallas as pl


def kernel(x, recording, walls):
    raise NotImplementedError("write your pallas kernel here")


# speedup vs baseline: 1.0942x; 1.0942x over previous
import functools

import jax
import jax.numpy as jnp
from jax.experimental import pallas as pl
from jax.experimental.pallas import tpu as pltpu

_K = 5
_MID = (_K * _K) // 2


def _unfold_circ(x, k):
    p = k // 2
    xp = jnp.pad(x, ((0, 0), (0, 0), (p, p), (p, p)), mode='wrap')
    b, c, h, w = x.shape
    patches = jnp.stack(
        [xp[:, :, i:i + h, j:j + w] for i in range(k) for j in range(k)],
        axis=2)
    return patches.reshape(b, c * k * k, h * w)


def _argmin_body(q_ref, kt_ref, out_ref, minv, argv, *, tm, nm):
    m = pl.program_id(0)

    @pl.when(m == 0)
    def _init():
        minv[...] = jnp.full(minv.shape, jnp.inf, jnp.float32)
        argv[...] = jnp.zeros(argv.shape, jnp.int32)

    kt = kt_ref[...]
    q = q_ref[...]
    b2 = jnp.sum(kt * kt, axis=0, keepdims=True)
    a2 = jnp.sum(q * q, axis=1, keepdims=True)
    ab = jnp.dot(q, kt, preferred_element_type=jnp.float32)
    s = jnp.maximum(a2 + b2 - 2.0 * ab, 0.0)
    lmin = jnp.min(s, axis=1, keepdims=True)
    larg = jnp.argmin(s, axis=1).astype(jnp.int32)[:, None] + m * tm
    better = lmin < minv[...]
    argv[...] = jnp.where(better, larg, argv[...])
    minv[...] = jnp.where(better, lmin, minv[...])

    @pl.when(m == nm - 1)
    def _fin():
        out_ref[...] = argv[...]


def _nn_argmin(q, kt, tm=512):
    n, kd = q.shape
    m = kt.shape[1]
    nm = m // tm
    return pl.pallas_call(
        functools.partial(_argmin_body, tm=tm, nm=nm),
        grid=(nm,),
        in_specs=[
            pl.BlockSpec((n, kd), lambda i: (0, 0)),
            pl.BlockSpec((kd, tm), lambda i: (0, i)),
        ],
        out_specs=pl.BlockSpec((n, 1), lambda i: (0, 0)),
        out_shape=jax.ShapeDtypeStruct((n, 1), jnp.int32),
        scratch_shapes=[
            pltpu.VMEM((n, 1), jnp.float32),
            pltpu.VMEM((n, 1), jnp.int32),
        ],
    )(q, kt)


def kernel(x, recording, walls):
    k = _K
    b, d = x.shape[0], x.shape[1]
    t = recording.shape[0]
    h, w = walls.shape
    hw = h * w

    w4 = jnp.broadcast_to(walls[None, None], (t, 1, h, w))
    rec = jnp.concatenate([recording, w4], axis=1)
    cn = rec.shape[1]
    dim = cn * k * k

    unf = _unfold_circ(rec, k)
    rows = jnp.transpose(unf, (0, 2, 1)).reshape(t * hw, dim)
    base = rows[:-hw]
    mkeys = base.shape[0]

    kd = 256
    kt = jnp.pad(base, ((0, 0), (0, kd - dim))).T

    rec_center = jnp.transpose(
        rec[1:].reshape(t - 1, cn, hw), (0, 2, 1)).reshape((t - 1) * hw, cn)

    wrep = jnp.broadcast_to(walls[None, None, None], (b, d, 1, h, w))
    xc = jnp.concatenate([x, wrep], axis=2)

    cur = xc[:, 0]
    steps = [cur]
    losses = []
    for i in range(1, d):
        qu = _unfold_circ(cur, k)
        q = jnp.transpose(qu, (0, 2, 1)).reshape(b * hw, dim)
        q = jnp.pad(q, ((0, 0), (0, kd - dim)))
        idx = _nn_argmin(q, kt).reshape(b, hw)
        val = jnp.take(rec_center, idx, axis=0)
        val = jnp.transpose(val, (0, 2, 1)).reshape(b, cn, h, w)
        losses.append(jnp.mean((val - xc[:, i]) ** 2))
        steps.append(val)
        cur = val

    all_steps = jnp.stack(steps, axis=1)[:, :, :-1]
    return all_steps, jnp.stack(losses).mean()


# speedup vs baseline: 1.3243x; 1.2102x over previous
import jax
import jax.numpy as jnp
from jax.experimental import pallas as pl
from jax.experimental.pallas import tpu as pltpu

_K = 5
_TM = 512


def _unfold_circ(x, k):
    p = k // 2
    xp = jnp.pad(x, ((0, 0), (0, 0), (p, p), (p, p)), mode='wrap')
    b, c, h, w = x.shape
    patches = jnp.stack(
        [xp[:, :, i:i + h, j:j + w] for i in range(k) for j in range(k)],
        axis=2)
    return patches.reshape(b, c * k * k, h * w)


def _argmin_body(qt_ref, kt_ref, out_ref, q_scr, a2_scr, rmin_scr, rtile_scr,
                 *, nm):
    m = pl.program_id(0)
    n = q_scr.shape[0]

    @pl.when(m == 0)
    def _build_q():
        nb = qt_ref.shape[0]
        hw = qt_ref.shape[2]
        for b in range(nb):
            q_scr[b * hw:(b + 1) * hw, :] = jnp.transpose(qt_ref[b])
        q = q_scr[...]
        a2_scr[...] = jnp.sum(q * q, axis=1, keepdims=True)

    kt = kt_ref[0]
    b2 = jnp.sum(kt * kt, axis=0, keepdims=True)
    ab = jnp.dot(q_scr[...], kt, preferred_element_type=jnp.float32)
    s = a2_scr[...] + b2 - 2.0 * ab

    @pl.when(m == 0)
    def _init():
        rmin_scr[...] = s
        rtile_scr[...] = jnp.zeros(s.shape, jnp.int32)

    @pl.when(m > 0)
    def _merge():
        upd = s < rmin_scr[...]
        rmin_scr[...] = jnp.where(upd, s, rmin_scr[...])
        rtile_scr[...] = jnp.where(upd, m, rtile_scr[...])

    @pl.when(m == nm - 1)
    def _extract():
        rclamp = jnp.maximum(rmin_scr[...], 0.0)
        vmin = jnp.min(rclamp, axis=1, keepdims=True)
        lane = jax.lax.broadcasted_iota(jnp.int32, (n, _TM), 1)
        gcand = jnp.where(rclamp == vmin,
                          rtile_scr[...] * _TM + lane, jnp.int32(2**30))
        out_ref[...] = jnp.min(gcand, axis=1, keepdims=True)


def _nn_argmin(qt, kt):
    import functools
    nb, kd, hw = qt.shape
    ntb = kt.shape[0]
    tiles_per_blk = hw // _TM
    nm = ntb * tiles_per_blk
    n = nb * hw
    return pl.pallas_call(
        functools.partial(_argmin_body, nm=nm),
        grid=(nm,),
        in_specs=[
            pl.BlockSpec((nb, kd, hw), lambda m: (0, 0, 0)),
            pl.BlockSpec((1, kd, _TM),
                         lambda m: (m // tiles_per_blk, 0, m % tiles_per_blk)),
        ],
        out_specs=pl.BlockSpec((n, 1), lambda m: (0, 0)),
        out_shape=jax.ShapeDtypeStruct((n, 1), jnp.int32),
        scratch_shapes=[
            pltpu.VMEM((n, kd), jnp.float32),
            pltpu.VMEM((n, 1), jnp.float32),
            pltpu.VMEM((n, _TM), jnp.float32),
            pltpu.VMEM((n, _TM), jnp.int32),
        ],
    )(qt, kt)


def kernel(x, recording, walls):
    k = _K
    b, d = x.shape[0], x.shape[1]
    t = recording.shape[0]
    h, w = walls.shape
    hw = h * w

    w4 = jnp.broadcast_to(walls[None, None], (t, 1, h, w))
    rec = jnp.concatenate([recording, w4], axis=1)
    cn = rec.shape[1]
    dim = cn * k * k
    kd = 256

    unfr = _unfold_circ(rec, k)
    ktb = jnp.pad(unfr[:t - 1], ((0, 0), (0, kd - dim), (0, 0)))

    rec_center = jnp.transpose(
        rec[1:].reshape(t - 1, cn, hw), (0, 2, 1)).reshape((t - 1) * hw, cn)

    wrep = jnp.broadcast_to(walls[None, None, None], (b, d, 1, h, w))
    xc = jnp.concatenate([x, wrep], axis=2)

    cur = xc[:, 0]
    steps = [cur]
    losses = []
    for i in range(1, d):
        unfq = _unfold_circ(cur, k)
        qt = jnp.pad(unfq, ((0, 0), (0, kd - dim), (0, 0)))
        idx = _nn_argmin(qt, ktb).reshape(b, hw)
        val = jnp.take(rec_center, idx, axis=0)
        val = jnp.transpose(val, (0, 2, 1)).reshape(b, cn, h, w)
        losses.append(jnp.mean((val - xc[:, i]) ** 2))
        steps.append(val)
        cur = val

    all_steps = jnp.stack(steps, axis=1)[:, :, :-1]
    return all_steps, jnp.stack(losses).mean()


# speedup vs baseline: 1.3779x; 1.0405x over previous
import jax
import jax.numpy as jnp
from jax.experimental import pallas as pl
from jax.experimental.pallas import tpu as pltpu

_K = 5
_TM = 512


def _unfold_circ(x, k):
    p = k // 2
    xp = jnp.pad(x, ((0, 0), (0, 0), (p, p), (p, p)), mode='wrap')
    b, c, h, w = x.shape
    patches = jnp.stack(
        [xp[:, :, i:i + h, j:j + w] for i in range(k) for j in range(k)],
        axis=2)
    return patches.reshape(b, c * k * k, h * w)


def _argmin_body(qt_ref, kt_ref, out_ref, q_scr, a2_scr, rmin_scr, rtile_scr,
                 *, nm):
    m = pl.program_id(0)
    n = q_scr.shape[0]

    @pl.when(m == 0)
    def _build_q():
        nb = qt_ref.shape[0]
        hw = qt_ref.shape[2]
        for b in range(nb):
            q_scr[b * hw:(b + 1) * hw, :] = jnp.transpose(qt_ref[b])
        q = q_scr[...]
        a2_scr[...] = jnp.sum(q * q, axis=1, keepdims=True)

    kt = kt_ref[0]
    b2 = jnp.sum(kt * kt, axis=0, keepdims=True)
    ab = jnp.dot(q_scr[...], kt, preferred_element_type=jnp.float32)
    s = a2_scr[...] + b2 - 2.0 * ab

    @pl.when(m == 0)
    def _init():
        rmin_scr[...] = s
        rtile_scr[...] = jnp.zeros(s.shape, jnp.int32)

    @pl.when(m > 0)
    def _merge():
        upd = s < rmin_scr[...]
        rmin_scr[...] = jnp.where(upd, s, rmin_scr[...])
        rtile_scr[...] = jnp.where(upd, m, rtile_scr[...])

    @pl.when(m == nm - 1)
    def _extract():
        rclamp = jnp.maximum(rmin_scr[...], 0.0)
        vmin = jnp.min(rclamp, axis=1, keepdims=True)
        lane = jax.lax.broadcasted_iota(jnp.int32, (n, _TM), 1)
        gcand = jnp.where(rclamp == vmin,
                          rtile_scr[...] * _TM + lane, jnp.int32(2**30))
        out_ref[...] = jnp.min(gcand, axis=1, keepdims=True)


def _nn_argmin(qt, kt):
    import functools
    nb, kd, hw = qt.shape
    ntb = kt.shape[0]
    tiles_per_blk = hw // _TM
    nm = ntb * tiles_per_blk
    n = nb * hw
    return pl.pallas_call(
        functools.partial(_argmin_body, nm=nm),
        grid=(nm,),
        in_specs=[
            pl.BlockSpec((nb, kd, hw), lambda m: (0, 0, 0)),
            pl.BlockSpec((1, kd, _TM),
                         lambda m: (m // tiles_per_blk, 0, m % tiles_per_blk)),
        ],
        out_specs=pl.BlockSpec((n, 1), lambda m: (0, 0)),
        out_shape=jax.ShapeDtypeStruct((n, 1), jnp.int32),
        scratch_shapes=[
            pltpu.VMEM((n, kd), jnp.float32),
            pltpu.VMEM((n, 1), jnp.float32),
            pltpu.VMEM((n, _TM), jnp.float32),
            pltpu.VMEM((n, _TM), jnp.int32),
        ],
    )(qt, kt)


def kernel(x, recording, walls):
    k = _K
    b, d = x.shape[0], x.shape[1]
    t = recording.shape[0]
    h, w = walls.shape
    hw = h * w

    w4 = jnp.broadcast_to(walls[None, None], (t, 1, h, w))
    rec = jnp.concatenate([recording, w4], axis=1)
    cn = rec.shape[1]
    dim = cn * k * k
    kd = 256

    unfr = _unfold_circ(rec, k)
    ktb = unfr[:t - 1]

    rec_center = jnp.transpose(
        rec[1:].reshape(t - 1, cn, hw), (0, 2, 1)).reshape((t - 1) * hw, cn)

    wrep = jnp.broadcast_to(walls[None, None, None], (b, d, 1, h, w))
    xc = jnp.concatenate([x, wrep], axis=2)

    cur = xc[:, 0]
    steps = [cur]
    losses = []
    for i in range(1, d):
        qt = _unfold_circ(cur, k)
        idx = _nn_argmin(qt, ktb).reshape(b, hw)
        val = jnp.take(rec_center, idx, axis=0)
        val = jnp.transpose(val, (0, 2, 1)).reshape(b, cn, h, w)
        losses.append(jnp.mean((val - xc[:, i]) ** 2))
        steps.append(val)
        cur = val

    all_steps = jnp.stack(steps, axis=1)[:, :, :-1]
    return all_steps, jnp.stack(losses).mean()


# speedup vs baseline: 2.3764x; 1.7246x over previous
import functools

import jax
import jax.numpy as jnp
from jax.experimental import pallas as pl
from jax.experimental.pallas import tpu as pltpu

_K = 5
_P = _K // 2
_TM = 512
_KD = 256


def _rot(a, s):
    s = s % a.shape[-1]
    if s == 0:
        return a
    return jnp.concatenate([a[..., s:], a[..., :s]], axis=-1)


def _shifted(img, xmod, dy, dx, w):
    imy = _rot(img, w * dy)
    if dx == 0:
        return imy
    if dx > 0:
        return jnp.where(xmod < w - dx, _rot(imy, dx), _rot(imy, dx - w))
    return jnp.where(xmod >= -dx, _rot(imy, dx), _rot(imy, dx + w))


def _build_patch_rows(img_ref, nimg, cn, hw, w, write_row):
    img = img_ref[...].reshape(nimg * cn, hw)
    xmod = jax.lax.broadcasted_iota(jnp.int32, (nimg * cn, hw), 1) % w
    for dy in range(-_P, _P + 1):
        for dx in range(-_P, _P + 1):
            o = (dy + _P) * _K + (dx + _P)
            sh = _shifted(img, xmod, dy, dx, w)
            blk = sh.reshape(nimg, cn, hw)
            for c in range(cn):
                write_row(c * _K * _K + o, blk[:, c, :])


def _base_body(rec_ref, out_ref):
    t, cn, hw = rec_ref.shape
    w = 32

    def write_row(r, blk):
        for tt in range(t - 1):
            out_ref[tt, r, :] = blk[tt]

    _build_patch_rows(rec_ref, t, cn, hw, w, write_row)
    out_ref[:, cn * _K * _K:, :] = jnp.zeros(
        (t - 1, _KD - cn * _K * _K, hw), jnp.float32)


def _build_base(rec):
    t, cn, hw = rec.shape
    return pl.pallas_call(
        _base_body,
        out_shape=jax.ShapeDtypeStruct((t - 1, _KD, hw), jnp.float32),
    )(rec)


def _argmin_body(cur_ref, kt_ref, out_ref, qt_scr, q_scr, a2_scr, rmin_scr,
                 rtile_scr, *, nm):
    m = pl.program_id(0)
    n = q_scr.shape[0]

    @pl.when(m == 0)
    def _build_q():
        nb, cn, hw = cur_ref.shape
        w = 32
        dim = cn * _K * _K

        def write_row(r, blk):
            for b in range(nb):
                qt_scr[b, r, :] = blk[b]

        _build_patch_rows(cur_ref, nb, cn, hw, w, write_row)
        qt_scr[:, dim:, :] = jnp.zeros((nb, _KD - dim, hw), jnp.float32)
        for b in range(nb):
            q_scr[b * hw:(b + 1) * hw, :] = jnp.transpose(qt_scr[b])
        q = q_scr[...]
        a2_scr[...] = jnp.sum(q * q, axis=1, keepdims=True)

    kt = kt_ref[0]
    b2 = jnp.sum(kt * kt, axis=0, keepdims=True)
    ab = jnp.dot(q_scr[...], kt, preferred_element_type=jnp.float32)
    s = a2_scr[...] + b2 - 2.0 * ab

    @pl.when(m == 0)
    def _init():
        rmin_scr[...] = s
        rtile_scr[...] = jnp.zeros(s.shape, jnp.int32)

    @pl.when(m > 0)
    def _merge():
        upd = s < rmin_scr[...]
        rmin_scr[...] = jnp.where(upd, s, rmin_scr[...])
        rtile_scr[...] = jnp.where(upd, m, rtile_scr[...])

    @pl.when(m == nm - 1)
    def _extract():
        rclamp = jnp.maximum(rmin_scr[...], 0.0)
        vmin = jnp.min(rclamp, axis=1, keepdims=True)
        lane = jax.lax.broadcasted_iota(jnp.int32, (n, _TM), 1)
        gcand = jnp.where(rclamp == vmin,
                          rtile_scr[...] * _TM + lane, jnp.int32(2**30))
        out_ref[...] = jnp.min(gcand, axis=1, keepdims=True)


def _nn_argmin(cur, ktmat):
    nb, cn, hw = cur.shape
    ntb = ktmat.shape[0]
    tiles_per_blk = hw // _TM
    nm = ntb * tiles_per_blk
    n = nb * hw
    return pl.pallas_call(
        functools.partial(_argmin_body, nm=nm),
        grid=(nm,),
        in_specs=[
            pl.BlockSpec((nb, cn, hw), lambda m: (0, 0, 0)),
            pl.BlockSpec((1, _KD, _TM),
                         lambda m: (m // tiles_per_blk, 0, m % tiles_per_blk)),
        ],
        out_specs=pl.BlockSpec((n, 1), lambda m: (0, 0)),
        out_shape=jax.ShapeDtypeStruct((n, 1), jnp.int32),
        scratch_shapes=[
            pltpu.VMEM((nb, _KD, hw), jnp.float32),
            pltpu.VMEM((n, _KD), jnp.float32),
            pltpu.VMEM((n, 1), jnp.float32),
            pltpu.VMEM((n, _TM), jnp.float32),
            pltpu.VMEM((n, _TM), jnp.int32),
        ],
    )(cur, ktmat)


def kernel(x, recording, walls):
    b, d = x.shape[0], x.shape[1]
    t = recording.shape[0]
    h, w = walls.shape
    hw = h * w

    w4 = jnp.broadcast_to(walls[None, None], (t, 1, h, w))
    rec = jnp.concatenate([recording, w4], axis=1).reshape(t, -1, hw)
    cn = rec.shape[1]

    ktmat = _build_base(rec)

    rec_center = jnp.transpose(
        rec[1:], (0, 2, 1)).reshape((t - 1) * hw, cn)

    wrep = jnp.broadcast_to(walls[None, None, None], (b, d, 1, h, w))
    xc = jnp.concatenate([x, wrep], axis=2)

    cur = xc[:, 0].reshape(b, cn, hw)
    steps = [xc[:, 0]]
    losses = []
    for i in range(1, d):
        idx = _nn_argmin(cur, ktmat).reshape(b, hw)
        val = jnp.take(rec_center, idx, axis=0)
        cur = jnp.transpose(val, (0, 2, 1))
        val_img = cur.reshape(b, cn, h, w)
        losses.append(jnp.mean((val_img - xc[:, i]) ** 2))
        steps.append(val_img)

    all_steps = jnp.stack(steps, axis=1)[:, :, :-1]
    return all_steps, jnp.stack(losses).mean()
